# baseline (speedup 1.0000x reference)
import jax
import jax.numpy as jnp
from jax.experimental import pallas as pl
from jax.experimental.pallas import tpu as pltpu

P = 1000
M = 64
R = 32
BLK = 200


def _body(alpha_ref, beta_ref, brel_ref, wr_ref, batt_ref,
          xyb_ref, nei_ref, hid_ref, wh_ref, wn_ref, out_ref):
    i = pl.program_id(0)
    x = xyb_ref[:, 0, :]
    y = xyb_ref[:, 1, :]
    hid = hid_ref[...]

    acc = jnp.zeros((BLK, P), dtype=jnp.float32)
    for r in range(R):
        zr = x * alpha_ref[r] + y * beta_ref[r] + brel_ref[r]
        acc = acc + jnp.maximum(zr, 0.0) * wr_ref[r]

    hrow = hid_ref[pl.ds(i * BLK, BLK), :]
    a = jnp.sum(hrow * wh_ref[...], axis=1, keepdims=True)
    c = jnp.sum(hid * wn_ref[...], axis=1, keepdims=True)
    c_row = c.reshape(1, P)

    z = acc + a + c_row + batt_ref[0]
    mask = nei_ref[...] > 0
    zq = jnp.where(mask & (z != 0.0), z, -1e-6)

    m = jnp.max(zq, axis=1, keepdims=True)
    e = jnp.exp(zq - m)
    d = jnp.sum(e, axis=1, keepdims=True)
    p = jnp.where(mask, e / d, 0.0)

    out_ref[...] = jnp.dot(p, hid, preferred_element_type=jnp.float32)


@jax.jit
def kernel(hidden_state, corr_index, nei_index, W_rel, b_rel, W_att, b_att):
    xyb = jax.lax.transpose(corr_index, (0, 2, 1))
    nei = nei_index.astype(jnp.int32)
    alpha = W_rel[:, 0]
    beta = W_rel[:, 1]
    wr = W_att[0, :R]
    wh = W_att[0, R:R + M].reshape(1, M)
    wn = W_att[0, R + M:].reshape(1, M)

    grid = P // BLK
    return pl.pallas_call(
        _body,
        grid=(grid,),
        in_specs=[
            pl.BlockSpec(memory_space=pltpu.SMEM),
            pl.BlockSpec(memory_space=pltpu.SMEM),
            pl.BlockSpec(memory_space=pltpu.SMEM),
            pl.BlockSpec(memory_space=pltpu.SMEM),
            pl.BlockSpec(memory_space=pltpu.SMEM),
            pl.BlockSpec((BLK, 2, P), lambda i: (i, 0, 0)),
            pl.BlockSpec((BLK, P), lambda i: (i, 0)),
            pl.BlockSpec((P, M), lambda i: (0, 0)),
            pl.BlockSpec((1, M), lambda i: (0, 0)),
            pl.BlockSpec((1, M), lambda i: (0, 0)),
        ],
        out_specs=pl.BlockSpec((BLK, M), lambda i: (i, 0)),
        out_shape=jax.ShapeDtypeStruct((P, M), jnp.float32),
        compiler_params=pltpu.CompilerParams(
            dimension_semantics=("arbitrary",),
        ),
    )(alpha, beta, b_rel, wr, b_att, xyb, nei, hidden_state, wh, wn)


# speedup vs baseline: 2.9980x; 2.9980x over previous
import jax
import jax.numpy as jnp
from jax.experimental import pallas as pl
from jax.experimental.pallas import tpu as pltpu

P = 1000
M = 64
R = 32
BLK = 200


def _body(alpha_ref, beta_ref, brel_ref, wr_ref, batt_ref,
          xyb_ref, nei_ref, hid_ref, wh_ref, wn_ref, out_ref,
          xs_ref, ys_ref, sem_x, sem_y):
    i = pl.program_id(0)
    cx = pltpu.make_async_copy(xyb_ref.at[:, 0], xs_ref, sem_x)
    cy = pltpu.make_async_copy(xyb_ref.at[:, 1], ys_ref, sem_y)
    cx.start()
    cy.start()
    hid = hid_ref[...]
    cx.wait()
    cy.wait()
    x = xs_ref[...]
    y = ys_ref[...]

    acc = jnp.zeros((BLK, P), dtype=jnp.float32)
    for r in range(R):
        zr = x * alpha_ref[r] + y * beta_ref[r] + brel_ref[r]
        acc = acc + jnp.maximum(zr, 0.0) * wr_ref[r]

    hrow = hid_ref[pl.ds(i * BLK, BLK), :]
    a = jnp.sum(hrow * wh_ref[...], axis=1, keepdims=True)
    c = jnp.sum(hid * wn_ref[...], axis=1, keepdims=True)
    c_row = c.reshape(1, P)

    z = acc + a + c_row + batt_ref[0]
    mask = nei_ref[...] > 0
    zq = jnp.where(mask & (z != 0.0), z, -1e-6)

    m = jnp.max(zq, axis=1, keepdims=True)
    e = jnp.exp(zq - m)
    d = jnp.sum(e, axis=1, keepdims=True)
    p = jnp.where(mask, e / d, 0.0)

    out_ref[...] = jnp.dot(p, hid, preferred_element_type=jnp.float32)


@jax.jit
def kernel(hidden_state, corr_index, nei_index, W_rel, b_rel, W_att, b_att):
    xyb = jax.lax.transpose(corr_index, (0, 2, 1))
    nei = nei_index.astype(jnp.int32)
    alpha = W_rel[:, 0]
    beta = W_rel[:, 1]
    wr = W_att[0, :R]
    wh = W_att[0, R:R + M].reshape(1, M)
    wn = W_att[0, R + M:].reshape(1, M)

    grid = P // BLK
    return pl.pallas_call(
        _body,
        grid=(grid,),
        in_specs=[
            pl.BlockSpec(memory_space=pltpu.SMEM),
            pl.BlockSpec(memory_space=pltpu.SMEM),
            pl.BlockSpec(memory_space=pltpu.SMEM),
            pl.BlockSpec(memory_space=pltpu.SMEM),
            pl.BlockSpec(memory_space=pltpu.SMEM),
            pl.BlockSpec((BLK, 2, P), lambda i: (i, 0, 0)),
            pl.BlockSpec((BLK, P), lambda i: (i, 0)),
            pl.BlockSpec((P, M), lambda i: (0, 0)),
            pl.BlockSpec((1, M), lambda i: (0, 0)),
            pl.BlockSpec((1, M), lambda i: (0, 0)),
        ],
        out_specs=pl.BlockSpec((BLK, M), lambda i: (i, 0)),
        out_shape=jax.ShapeDtypeStruct((P, M), jnp.float32),
        scratch_shapes=[
            pltpu.VMEM((BLK, P), jnp.float32),
            pltpu.VMEM((BLK, P), jnp.float32),
            pltpu.SemaphoreType.DMA,
            pltpu.SemaphoreType.DMA,
        ],
        compiler_params=pltpu.CompilerParams(
            dimension_semantics=("arbitrary",),
        ),
    )(alpha, beta, b_rel, wr, b_att, xyb, nei, hidden_state, wh, wn)
